# baseline (speedup 1.0000x reference)
import dataclasses
import functools

import numpy as np
import jax
import jax.numpy as jnp
from jax import lax
from jax.experimental import pallas as pl
from jax.experimental.pallas import tpu as pltpu
from jax.experimental.pallas import tpu_sc as plsc

N = 10000
E = 160000
D_IN = 256
H = 512
BN_SCALE = np.float32(1.0 / np.sqrt(1.0 + 1e-5))

NC = 2
NS = 16
HC = 128
NCHUNK = H // HC
NP = 10240
EB = 80
E_PT = E // NS
ITERS = E_PT // EB
ROWS_PT = NP // NS
DEG_PT = E // (NC * NS)
DEG_FULL = DEG_PT // 16

ROW_BLK = 400
GRID = N // ROW_BLK

_MESH = plsc.VectorSubcoreMesh(core_axis_name="c", subcore_axis_name="s")

_SC_PARAMS = pltpu.CompilerParams()
if "needs_layout_passes" in pltpu.CompilerParams.__dataclass_fields__:
    _SC_PARAMS = dataclasses.replace(_SC_PARAMS, needs_layout_passes=False)
if "use_tc_tiling_on_sc" in pltpu.CompilerParams.__dataclass_fields__:
    _SC_PARAMS = dataclasses.replace(_SC_PARAMS, use_tc_tiling_on_sc=False)


def _deg_body(col_ref, out_ref, colbuf, partial, sem):
    del sem
    c = lax.axis_index("c")
    s = lax.axis_index("s")
    w = s * NC + c

    zeros16 = jnp.zeros((16,), jnp.float32)

    @pl.loop(0, N // 16)
    def _(i):
        partial[pl.ds(i * 16, 16)] = zeros16

    colbuf[pl.ds(DEG_FULL * 16, 16)] = jnp.zeros((16,), jnp.int32)
    pltpu.sync_copy(col_ref.at[w, 0], colbuf.at[pl.ds(0, DEG_PT)])

    ones16 = jnp.ones((16,), jnp.float32)

    @pl.loop(0, DEG_FULL)
    def _(i):
        idx = colbuf[pl.ds(i * 16, 16)]
        plsc.addupdate_scatter(partial, [idx], ones16)

    idx = colbuf[pl.ds(DEG_FULL * 16, 16)]
    mask = lax.iota(jnp.int32, 16) < (DEG_PT - DEG_FULL * 16)
    plsc.addupdate_scatter(partial, [idx], ones16, mask=mask)

    pltpu.sync_copy(partial, out_ref.at[w, 0])


@jax.jit
def _deg_call(col3d):
    k = pl.kernel(
        _deg_body,
        out_type=jax.ShapeDtypeStruct((NC * NS, 1, N), jnp.float32),
        mesh=_MESH,
        compiler_params=_SC_PARAMS,
        scratch_types=[
            pltpu.VMEM((DEG_FULL * 16 + 16,), jnp.int32),
            pltpu.VMEM((N,), jnp.float32),
            pltpu.SemaphoreType.DMA,
        ],
    )
    return k(col3d)


def _scatter_body(g_ref, row_ref, col_ref, z_ref, out_ref, acc,
                  rows_a, rows_b, ridx, cidx, sem_a, sem_b):
    c = lax.axis_index("c")
    s = lax.axis_index("s")

    pltpu.sync_copy(row_ref.at[s], ridx)
    pltpu.sync_copy(col_ref.at[s], cidx)

    for jj in range(NCHUNK // NC):
        j = c * (NCHUNK // NC) + jj
        gj = g_ref.at[j]

        pltpu.sync_copy(z_ref, acc.at[pl.ds(s * ROWS_PT, ROWS_PT)])

        plsc.subcore_barrier()

        pltpu.async_copy(gj.at[ridx.at[0]], rows_a, sem_a)

        @pl.loop(0, (ITERS - 1) // 2)
        def _(h):
            it = h * 2
            cp_b = pltpu.async_copy(gj.at[ridx.at[it + 1]], rows_b, sem_b)
            pltpu.make_async_copy(gj.at[ridx.at[it]], rows_a, sem_a).wait()
            pltpu.sync_copy(rows_a, acc.at[cidx.at[it]], add=True)
            pltpu.async_copy(gj.at[ridx.at[it + 2]], rows_a, sem_a)
            cp_b.wait()
            pltpu.sync_copy(rows_b, acc.at[cidx.at[it + 1]], add=True)

        pltpu.make_async_copy(gj.at[ridx.at[ITERS - 1]], rows_a, sem_a).wait()
        pltpu.sync_copy(rows_a, acc.at[cidx.at[ITERS - 1]], add=True)
        plsc.subcore_barrier()

        pltpu.sync_copy(acc.at[pl.ds(s * ROWS_PT, ROWS_PT)],
                        out_ref.at[j, pl.ds(s * ROWS_PT, ROWS_PT)])


@jax.jit
def _scatter_call(g4, row3, col3, zrows):
    k = pl.kernel(
        _scatter_body,
        out_type=jax.ShapeDtypeStruct((NCHUNK, NP, HC), jnp.float32),
        mesh=_MESH,
        compiler_params=_SC_PARAMS,
        scratch_types=[
            pltpu.VMEM_SHARED((NP, HC), jnp.float32),
            pltpu.VMEM((EB, HC), jnp.float32),
            pltpu.VMEM((EB, HC), jnp.float32),
            pltpu.VMEM((ITERS, EB), jnp.int32),
            pltpu.VMEM((ITERS, EB), jnp.int32),
            pltpu.SemaphoreType.DMA,
            pltpu.SemaphoreType.DMA,
        ],
    )
    return k(g4, row3, col3, zrows)


def _dis_body(degp_ref, dis_ref):
    deg = jnp.sum(degp_ref[...], axis=0) + 1.0
    dis_ref[...] = lax.rsqrt(deg)[:, None]


@jax.jit
def _dis_call(degp):
    return pl.pallas_call(
        _dis_body,
        out_shape=jax.ShapeDtypeStruct((N, 1), jnp.float32),
    )(degp)


def _enc_body(x_ref, ew_ref, eb_ref, bg_ref, bb_ref, w1_ref, dis_ref,
              g_ref):
    h = jnp.dot(x_ref[...], ew_ref[...],
                preferred_element_type=jnp.float32,
                precision=lax.Precision.DEFAULT)
    h = jnp.maximum(h + eb_ref[...], 0.0)
    h = h * (bg_ref[...] * BN_SCALE) + bb_ref[...]
    hw = jnp.dot(h, w1_ref[...],
                 preferred_element_type=jnp.float32,
                 precision=lax.Precision.DEFAULT)
    g = hw * dis_ref[...]
    for j in range(NCHUNK):
        g_ref[j] = g[:, j * HC:(j + 1) * HC]


@jax.jit
def _enc_call(x, enc_W, enc_b, bn_gamma, bn_beta, W1, dis):
    full = lambda shape: pl.BlockSpec(shape, lambda i: (0,) * len(shape))
    return pl.pallas_call(
        _enc_body,
        grid=(GRID,),
        in_specs=[
            pl.BlockSpec((ROW_BLK, D_IN), lambda i: (i, 0)),
            full((D_IN, H)),
            full((1, H)),
            full((1, H)),
            full((1, H)),
            full((H, H)),
            pl.BlockSpec((ROW_BLK, 1), lambda i: (i, 0)),
        ],
        out_specs=pl.BlockSpec((NCHUNK, ROW_BLK, HC), lambda i: (0, i, 0)),
        out_shape=jax.ShapeDtypeStruct((NCHUNK, NP, HC), jnp.float32),
    )(x, enc_W, enc_b.reshape(1, H), bn_gamma.reshape(1, H),
      bn_beta.reshape(1, H), W1, dis)


def _mid_body(s_ref, gin_ref, dis_ref, b1_ref, w2_ref, g_ref):
    scat = jnp.concatenate(
        [s_ref[j] + gin_ref[j] for j in range(NCHUNK)], axis=-1)
    h = jnp.maximum(scat * dis_ref[...] + b1_ref[...], 0.0)
    hw = jnp.dot(h, w2_ref[...],
                 preferred_element_type=jnp.float32,
                 precision=lax.Precision.DEFAULT)
    g = hw * dis_ref[...]
    for j in range(NCHUNK):
        g_ref[j] = g[:, j * HC:(j + 1) * HC]


@jax.jit
def _mid_call(s4, g4, dis, b1, W2):
    return pl.pallas_call(
        _mid_body,
        grid=(GRID,),
        in_specs=[
            pl.BlockSpec((NCHUNK, ROW_BLK, HC), lambda i: (0, i, 0)),
            pl.BlockSpec((NCHUNK, ROW_BLK, HC), lambda i: (0, i, 0)),
            pl.BlockSpec((ROW_BLK, 1), lambda i: (i, 0)),
            pl.BlockSpec((1, H), lambda i: (0, 0)),
            pl.BlockSpec((H, H), lambda i: (0, 0)),
        ],
        out_specs=pl.BlockSpec((NCHUNK, ROW_BLK, HC), lambda i: (0, i, 0)),
        out_shape=jax.ShapeDtypeStruct((NCHUNK, NP, HC), jnp.float32),
    )(s4, g4, dis, b1.reshape(1, H), W2)


def _fin_body(s_ref, gin_ref, dis_ref, b2_ref, out_ref):
    scat = jnp.concatenate(
        [s_ref[j] + gin_ref[j] for j in range(NCHUNK)], axis=-1)
    out_ref[...] = scat * dis_ref[...] + b2_ref[...]


@jax.jit
def _fin_call(s4, g4, dis, b2):
    return pl.pallas_call(
        _fin_body,
        grid=(GRID,),
        in_specs=[
            pl.BlockSpec((NCHUNK, ROW_BLK, HC), lambda i: (0, i, 0)),
            pl.BlockSpec((NCHUNK, ROW_BLK, HC), lambda i: (0, i, 0)),
            pl.BlockSpec((ROW_BLK, 1), lambda i: (i, 0)),
            pl.BlockSpec((1, H), lambda i: (0, 0)),
        ],
        out_specs=pl.BlockSpec((ROW_BLK, H), lambda i: (i, 0)),
        out_shape=jax.ShapeDtypeStruct((N, H), jnp.float32),
    )(s4, g4, dis, b2.reshape(1, H))


def kernel(x, edge_index, enc_W, enc_b, bn_gamma, bn_beta, W1, b1, W2, b2):
    ei = edge_index.astype(jnp.int32)
    col3d = ei[1].reshape(NC * NS, 1, DEG_PT)
    row3 = ei[0].reshape(NS, ITERS, EB)
    col3 = ei[1].reshape(NS, ITERS, EB)
    degp = _deg_call(col3d)
    dis = _dis_call(degp.reshape(NC * NS, N))
    g1 = _enc_call(x, enc_W, enc_b, bn_gamma, bn_beta, W1, dis)
    zrows = jnp.zeros((ROWS_PT, HC), jnp.float32)
    s1 = _scatter_call(g1, row3, col3, zrows)
    g2 = _mid_call(s1, g1, dis, b1, W2)
    s2 = _scatter_call(g2, row3, col3, zrows)
    return _fin_call(s2, g2, dis, b2)


# speedup vs baseline: 1.0566x; 1.0566x over previous
import dataclasses
import functools

import numpy as np
import jax
import jax.numpy as jnp
from jax import lax
from jax.experimental import pallas as pl
from jax.experimental.pallas import tpu as pltpu
from jax.experimental.pallas import tpu_sc as plsc

N = 10000
E = 160000
D_IN = 256
H = 512
BN_SCALE = np.float32(1.0 / np.sqrt(1.0 + 1e-5))

NC = 2
NS = 16
HC = 128
NCHUNK = H // HC
NP = 10240
EB = 80
E_PT = E // NS
ITERS = E_PT // EB
ROWS_PT = NP // NS
DEG_PT = E // (NC * NS)
DEG_FULL = DEG_PT // 16

ROW_BLK = 1000
GRID = N // ROW_BLK

_MESH = plsc.VectorSubcoreMesh(core_axis_name="c", subcore_axis_name="s")

_SC_PARAMS = pltpu.CompilerParams()
if "needs_layout_passes" in pltpu.CompilerParams.__dataclass_fields__:
    _SC_PARAMS = dataclasses.replace(_SC_PARAMS, needs_layout_passes=False)
if "use_tc_tiling_on_sc" in pltpu.CompilerParams.__dataclass_fields__:
    _SC_PARAMS = dataclasses.replace(_SC_PARAMS, use_tc_tiling_on_sc=False)


def _deg_body(col_ref, out_ref, colbuf, partial, sem):
    del sem
    c = lax.axis_index("c")
    s = lax.axis_index("s")
    w = s * NC + c

    zeros16 = jnp.zeros((16,), jnp.float32)

    @pl.loop(0, N // 16)
    def _(i):
        partial[pl.ds(i * 16, 16)] = zeros16

    colbuf[pl.ds(DEG_FULL * 16, 16)] = jnp.zeros((16,), jnp.int32)
    pltpu.sync_copy(col_ref.at[w, 0], colbuf.at[pl.ds(0, DEG_PT)])

    ones16 = jnp.ones((16,), jnp.float32)

    @pl.loop(0, DEG_FULL)
    def _(i):
        idx = colbuf[pl.ds(i * 16, 16)]
        plsc.addupdate_scatter(partial, [idx], ones16)

    idx = colbuf[pl.ds(DEG_FULL * 16, 16)]
    mask = lax.iota(jnp.int32, 16) < (DEG_PT - DEG_FULL * 16)
    plsc.addupdate_scatter(partial, [idx], ones16, mask=mask)

    pltpu.sync_copy(partial, out_ref.at[w, 0])


@jax.jit
def _deg_call(col3d):
    k = pl.kernel(
        _deg_body,
        out_type=jax.ShapeDtypeStruct((NC * NS, 1, N), jnp.float32),
        mesh=_MESH,
        compiler_params=_SC_PARAMS,
        scratch_types=[
            pltpu.VMEM((DEG_FULL * 16 + 16,), jnp.int32),
            pltpu.VMEM((N,), jnp.float32),
            pltpu.SemaphoreType.DMA,
        ],
    )
    return k(col3d)


def _scatter_body(g_ref, row_ref, col_ref, out_ref, acc,
                  rows_a, rows_b, ridx, cidx, sem_a, sem_b):
    c = lax.axis_index("c")
    s = lax.axis_index("s")

    pltpu.sync_copy(row_ref.at[s], ridx)
    pltpu.sync_copy(col_ref.at[s], cidx)

    zeros16 = jnp.zeros((16,), jnp.float32)

    for jj in range(NCHUNK // NC):
        j = c * (NCHUNK // NC) + jj
        gj = g_ref.at[j]

        @pl.loop(0, EB)
        def _(r):
            @pl.loop(0, HC // 16)
            def _(k):
                rows_b[r, pl.ds(k * 16, 16)] = zeros16

        @pl.loop(0, ROWS_PT // EB)
        def _(q):
            base = s * ROWS_PT + q * EB
            pltpu.sync_copy(rows_b, acc.at[pl.ds(base, EB)])

        plsc.subcore_barrier()

        pltpu.async_copy(gj.at[ridx.at[0]], rows_a, sem_a)

        @pl.loop(0, (ITERS - 1) // 2)
        def _(h):
            it = h * 2
            cp_b = pltpu.async_copy(gj.at[ridx.at[it + 1]], rows_b, sem_b)
            pltpu.make_async_copy(gj.at[ridx.at[it]], rows_a, sem_a).wait()
            pltpu.sync_copy(rows_a, acc.at[cidx.at[it]], add=True)
            pltpu.async_copy(gj.at[ridx.at[it + 2]], rows_a, sem_a)
            cp_b.wait()
            pltpu.sync_copy(rows_b, acc.at[cidx.at[it + 1]], add=True)

        pltpu.make_async_copy(gj.at[ridx.at[ITERS - 1]], rows_a, sem_a).wait()
        pltpu.sync_copy(rows_a, acc.at[cidx.at[ITERS - 1]], add=True)
        plsc.subcore_barrier()

        @pl.loop(0, ROWS_PT // EB)
        def _(q):
            base = s * ROWS_PT + q * EB
            pltpu.sync_copy(acc.at[pl.ds(base, EB)], rows_a)
            pltpu.sync_copy(rows_a, out_ref.at[j, pl.ds(base, EB)])


@jax.jit
def _scatter_call(g4, row3, col3):
    k = pl.kernel(
        _scatter_body,
        out_type=jax.ShapeDtypeStruct((NCHUNK, NP, HC), jnp.float32),
        mesh=_MESH,
        compiler_params=_SC_PARAMS,
        scratch_types=[
            pltpu.VMEM_SHARED((NP, HC), jnp.float32),
            pltpu.VMEM((EB, HC), jnp.float32),
            pltpu.VMEM((EB, HC), jnp.float32),
            pltpu.VMEM((ITERS, EB), jnp.int32),
            pltpu.VMEM((ITERS, EB), jnp.int32),
            pltpu.SemaphoreType.DMA,
            pltpu.SemaphoreType.DMA,
        ],
    )
    return k(g4, row3, col3)


def _dis_body(degp_ref, dis_ref):
    deg = jnp.sum(degp_ref[...], axis=0) + 1.0
    dis_ref[...] = lax.rsqrt(deg)[:, None]


@jax.jit
def _dis_call(degp):
    return pl.pallas_call(
        _dis_body,
        out_shape=jax.ShapeDtypeStruct((N, 1), jnp.float32),
    )(degp)


def _enc_body(x_ref, ew_ref, eb_ref, bg_ref, bb_ref, w1_ref, dis_ref,
              g_ref):
    h = jnp.dot(x_ref[...], ew_ref[...],
                preferred_element_type=jnp.float32,
                precision=lax.Precision.DEFAULT)
    h = jnp.maximum(h + eb_ref[...], 0.0)
    h = h * (bg_ref[...] * BN_SCALE) + bb_ref[...]
    hw = jnp.dot(h, w1_ref[...],
                 preferred_element_type=jnp.float32,
                 precision=lax.Precision.DEFAULT)
    g = hw * dis_ref[...]
    for j in range(NCHUNK):
        g_ref[j] = g[:, j * HC:(j + 1) * HC]


@jax.jit
def _enc_call(x, enc_W, enc_b, bn_gamma, bn_beta, W1, dis):
    full = lambda shape: pl.BlockSpec(shape, lambda i: (0,) * len(shape))
    return pl.pallas_call(
        _enc_body,
        grid=(GRID,),
        in_specs=[
            pl.BlockSpec((ROW_BLK, D_IN), lambda i: (i, 0)),
            full((D_IN, H)),
            full((1, H)),
            full((1, H)),
            full((1, H)),
            full((H, H)),
            pl.BlockSpec((ROW_BLK, 1), lambda i: (i, 0)),
        ],
        out_specs=pl.BlockSpec((NCHUNK, ROW_BLK, HC), lambda i: (0, i, 0)),
        out_shape=jax.ShapeDtypeStruct((NCHUNK, NP, HC), jnp.float32),
    )(x, enc_W, enc_b.reshape(1, H), bn_gamma.reshape(1, H),
      bn_beta.reshape(1, H), W1, dis)


def _mid_body(s_ref, gin_ref, dis_ref, b1_ref, w2_ref, g_ref):
    scat = jnp.concatenate(
        [s_ref[j] + gin_ref[j] for j in range(NCHUNK)], axis=-1)
    h = jnp.maximum(scat * dis_ref[...] + b1_ref[...], 0.0)
    hw = jnp.dot(h, w2_ref[...],
                 preferred_element_type=jnp.float32,
                 precision=lax.Precision.DEFAULT)
    g = hw * dis_ref[...]
    for j in range(NCHUNK):
        g_ref[j] = g[:, j * HC:(j + 1) * HC]


@jax.jit
def _mid_call(s4, g4, dis, b1, W2):
    return pl.pallas_call(
        _mid_body,
        grid=(GRID,),
        in_specs=[
            pl.BlockSpec((NCHUNK, ROW_BLK, HC), lambda i: (0, i, 0)),
            pl.BlockSpec((NCHUNK, ROW_BLK, HC), lambda i: (0, i, 0)),
            pl.BlockSpec((ROW_BLK, 1), lambda i: (i, 0)),
            pl.BlockSpec((1, H), lambda i: (0, 0)),
            pl.BlockSpec((H, H), lambda i: (0, 0)),
        ],
        out_specs=pl.BlockSpec((NCHUNK, ROW_BLK, HC), lambda i: (0, i, 0)),
        out_shape=jax.ShapeDtypeStruct((NCHUNK, NP, HC), jnp.float32),
    )(s4, g4, dis, b1.reshape(1, H), W2)


def _fin_body(s_ref, gin_ref, dis_ref, b2_ref, out_ref):
    scat = jnp.concatenate(
        [s_ref[j] + gin_ref[j] for j in range(NCHUNK)], axis=-1)
    out_ref[...] = scat * dis_ref[...] + b2_ref[...]


@jax.jit
def _fin_call(s4, g4, dis, b2):
    return pl.pallas_call(
        _fin_body,
        grid=(GRID,),
        in_specs=[
            pl.BlockSpec((NCHUNK, ROW_BLK, HC), lambda i: (0, i, 0)),
            pl.BlockSpec((NCHUNK, ROW_BLK, HC), lambda i: (0, i, 0)),
            pl.BlockSpec((ROW_BLK, 1), lambda i: (i, 0)),
            pl.BlockSpec((1, H), lambda i: (0, 0)),
        ],
        out_specs=pl.BlockSpec((ROW_BLK, H), lambda i: (i, 0)),
        out_shape=jax.ShapeDtypeStruct((N, H), jnp.float32),
    )(s4, g4, dis, b2.reshape(1, H))


def kernel(x, edge_index, enc_W, enc_b, bn_gamma, bn_beta, W1, b1, W2, b2):
    ei = edge_index.astype(jnp.int32)
    col3d = ei[1].reshape(NC * NS, 1, DEG_PT)
    row3 = ei[0].reshape(NS, ITERS, EB)
    col3 = ei[1].reshape(NS, ITERS, EB)
    degp = _deg_call(col3d)
    dis = _dis_call(degp.reshape(NC * NS, N))
    g1 = _enc_call(x, enc_W, enc_b, bn_gamma, bn_beta, W1, dis)
    s1 = _scatter_call(g1, row3, col3)
    g2 = _mid_call(s1, g1, dis, b1, W2)
    s2 = _scatter_call(g2, row3, col3)
    return _fin_call(s2, g2, dis, b2)


# speedup vs baseline: 1.0618x; 1.0049x over previous
import dataclasses
import functools

import numpy as np
import jax
import jax.numpy as jnp
from jax import lax
from jax.experimental import pallas as pl
from jax.experimental.pallas import tpu as pltpu
from jax.experimental.pallas import tpu_sc as plsc

N = 10000
E = 160000
D_IN = 256
H = 512
BN_SCALE = np.float32(1.0 / np.sqrt(1.0 + 1e-5))

NC = 2
NS = 16
HC = 128
NCHUNK = H // HC
NP = 10240
EB = 80
E_PT = E // NS
ITERS = E_PT // EB
ROWS_PT = NP // NS
DEG_PT = E // (NC * NS)
DEG_FULL = DEG_PT // 16

ROW_BLK = 2000
GRID = N // ROW_BLK

_MESH = plsc.VectorSubcoreMesh(core_axis_name="c", subcore_axis_name="s")

_SC_PARAMS = pltpu.CompilerParams()
if "needs_layout_passes" in pltpu.CompilerParams.__dataclass_fields__:
    _SC_PARAMS = dataclasses.replace(_SC_PARAMS, needs_layout_passes=False)
if "use_tc_tiling_on_sc" in pltpu.CompilerParams.__dataclass_fields__:
    _SC_PARAMS = dataclasses.replace(_SC_PARAMS, use_tc_tiling_on_sc=False)


def _deg_body(col_ref, out_ref, colbuf, partial, sem):
    del sem
    c = lax.axis_index("c")
    s = lax.axis_index("s")
    w = s * NC + c

    zeros16 = jnp.zeros((16,), jnp.float32)

    @pl.loop(0, N // 16)
    def _(i):
        partial[pl.ds(i * 16, 16)] = zeros16

    colbuf[pl.ds(DEG_FULL * 16, 16)] = jnp.zeros((16,), jnp.int32)
    pltpu.sync_copy(col_ref.at[w, 0], colbuf.at[pl.ds(0, DEG_PT)])

    ones16 = jnp.ones((16,), jnp.float32)

    @pl.loop(0, DEG_FULL)
    def _(i):
        idx = colbuf[pl.ds(i * 16, 16)]
        plsc.addupdate_scatter(partial, [idx], ones16)

    idx = colbuf[pl.ds(DEG_FULL * 16, 16)]
    mask = lax.iota(jnp.int32, 16) < (DEG_PT - DEG_FULL * 16)
    plsc.addupdate_scatter(partial, [idx], ones16, mask=mask)

    pltpu.sync_copy(partial, out_ref.at[w, 0])


@jax.jit
def _deg_call(col3d):
    k = pl.kernel(
        _deg_body,
        out_type=jax.ShapeDtypeStruct((NC * NS, 1, N), jnp.float32),
        mesh=_MESH,
        compiler_params=_SC_PARAMS,
        scratch_types=[
            pltpu.VMEM((DEG_FULL * 16 + 16,), jnp.int32),
            pltpu.VMEM((N,), jnp.float32),
            pltpu.SemaphoreType.DMA,
        ],
    )
    return k(col3d)


def _scatter_body(g_ref, row_ref, col_ref, out_ref, acc,
                  rows_a, rows_b, ridx, cidx, sem_a, sem_b):
    c = lax.axis_index("c")
    s = lax.axis_index("s")

    pltpu.sync_copy(row_ref.at[s], ridx)
    pltpu.sync_copy(col_ref.at[s], cidx)

    zeros16 = jnp.zeros((16,), jnp.float32)

    for jj in range(NCHUNK // NC):
        j = c * (NCHUNK // NC) + jj
        gj = g_ref.at[j]

        @pl.loop(0, EB)
        def _(r):
            @pl.loop(0, HC // 16)
            def _(k):
                rows_b[r, pl.ds(k * 16, 16)] = zeros16

        @pl.loop(0, ROWS_PT // EB)
        def _(q):
            base = s * ROWS_PT + q * EB
            pltpu.sync_copy(rows_b, acc.at[pl.ds(base, EB)])

        plsc.subcore_barrier()

        pltpu.async_copy(gj.at[ridx.at[0]], rows_a, sem_a)

        @pl.loop(0, (ITERS - 1) // 2)
        def _(h):
            it = h * 2
            cp_b = pltpu.async_copy(gj.at[ridx.at[it + 1]], rows_b, sem_b)
            pltpu.make_async_copy(gj.at[ridx.at[it]], rows_a, sem_a).wait()
            pltpu.sync_copy(rows_a, acc.at[cidx.at[it]], add=True)
            pltpu.async_copy(gj.at[ridx.at[it + 2]], rows_a, sem_a)
            cp_b.wait()
            pltpu.sync_copy(rows_b, acc.at[cidx.at[it + 1]], add=True)

        pltpu.make_async_copy(gj.at[ridx.at[ITERS - 1]], rows_a, sem_a).wait()
        pltpu.sync_copy(rows_a, acc.at[cidx.at[ITERS - 1]], add=True)
        plsc.subcore_barrier()

        @pl.loop(0, ROWS_PT // EB)
        def _(q):
            base = s * ROWS_PT + q * EB
            pltpu.sync_copy(acc.at[pl.ds(base, EB)], rows_a)
            pltpu.sync_copy(rows_a, out_ref.at[j, pl.ds(base, EB)])


@jax.jit
def _scatter_call(g4, row3, col3):
    k = pl.kernel(
        _scatter_body,
        out_type=jax.ShapeDtypeStruct((NCHUNK, NP, HC), jnp.float32),
        mesh=_MESH,
        compiler_params=_SC_PARAMS,
        scratch_types=[
            pltpu.VMEM_SHARED((NP, HC), jnp.float32),
            pltpu.VMEM((EB, HC), jnp.float32),
            pltpu.VMEM((EB, HC), jnp.float32),
            pltpu.VMEM((ITERS, EB), jnp.int32),
            pltpu.VMEM((ITERS, EB), jnp.int32),
            pltpu.SemaphoreType.DMA,
            pltpu.SemaphoreType.DMA,
        ],
    )
    return k(g4, row3, col3)


def _dis_body(degp_ref, dis_ref):
    deg = jnp.sum(degp_ref[...], axis=0) + 1.0
    dis_ref[...] = lax.rsqrt(deg)[:, None]


@jax.jit
def _dis_call(degp):
    return pl.pallas_call(
        _dis_body,
        out_shape=jax.ShapeDtypeStruct((N, 1), jnp.float32),
    )(degp)


def _enc_body(x_ref, ew_ref, eb_ref, bg_ref, bb_ref, w1_ref, dis_ref,
              g_ref):
    h = jnp.dot(x_ref[...], ew_ref[...],
                preferred_element_type=jnp.float32,
                precision=lax.Precision.DEFAULT)
    h = jnp.maximum(h + eb_ref[...], 0.0)
    h = h * (bg_ref[...] * BN_SCALE) + bb_ref[...]
    hw = jnp.dot(h, w1_ref[...],
                 preferred_element_type=jnp.float32,
                 precision=lax.Precision.DEFAULT)
    g = hw * dis_ref[...]
    for j in range(NCHUNK):
        g_ref[j] = g[:, j * HC:(j + 1) * HC]


@jax.jit
def _enc_call(x, enc_W, enc_b, bn_gamma, bn_beta, W1, dis):
    full = lambda shape: pl.BlockSpec(shape, lambda i: (0,) * len(shape))
    return pl.pallas_call(
        _enc_body,
        grid=(GRID,),
        in_specs=[
            pl.BlockSpec((ROW_BLK, D_IN), lambda i: (i, 0)),
            full((D_IN, H)),
            full((1, H)),
            full((1, H)),
            full((1, H)),
            full((H, H)),
            pl.BlockSpec((ROW_BLK, 1), lambda i: (i, 0)),
        ],
        out_specs=pl.BlockSpec((NCHUNK, ROW_BLK, HC), lambda i: (0, i, 0)),
        out_shape=jax.ShapeDtypeStruct((NCHUNK, NP, HC), jnp.float32),
    )(x, enc_W, enc_b.reshape(1, H), bn_gamma.reshape(1, H),
      bn_beta.reshape(1, H), W1, dis)


def _mid_body(s_ref, gin_ref, dis_ref, b1_ref, w2_ref, g_ref):
    scat = jnp.concatenate(
        [s_ref[j] + gin_ref[j] for j in range(NCHUNK)], axis=-1)
    h = jnp.maximum(scat * dis_ref[...] + b1_ref[...], 0.0)
    hw = jnp.dot(h, w2_ref[...],
                 preferred_element_type=jnp.float32,
                 precision=lax.Precision.DEFAULT)
    g = hw * dis_ref[...]
    for j in range(NCHUNK):
        g_ref[j] = g[:, j * HC:(j + 1) * HC]


@jax.jit
def _mid_call(s4, g4, dis, b1, W2):
    return pl.pallas_call(
        _mid_body,
        grid=(GRID,),
        in_specs=[
            pl.BlockSpec((NCHUNK, ROW_BLK, HC), lambda i: (0, i, 0)),
            pl.BlockSpec((NCHUNK, ROW_BLK, HC), lambda i: (0, i, 0)),
            pl.BlockSpec((ROW_BLK, 1), lambda i: (i, 0)),
            pl.BlockSpec((1, H), lambda i: (0, 0)),
            pl.BlockSpec((H, H), lambda i: (0, 0)),
        ],
        out_specs=pl.BlockSpec((NCHUNK, ROW_BLK, HC), lambda i: (0, i, 0)),
        out_shape=jax.ShapeDtypeStruct((NCHUNK, NP, HC), jnp.float32),
    )(s4, g4, dis, b1.reshape(1, H), W2)


def _fin_body(s_ref, gin_ref, dis_ref, b2_ref, out_ref):
    scat = jnp.concatenate(
        [s_ref[j] + gin_ref[j] for j in range(NCHUNK)], axis=-1)
    out_ref[...] = scat * dis_ref[...] + b2_ref[...]


@jax.jit
def _fin_call(s4, g4, dis, b2):
    return pl.pallas_call(
        _fin_body,
        grid=(GRID,),
        in_specs=[
            pl.BlockSpec((NCHUNK, ROW_BLK, HC), lambda i: (0, i, 0)),
            pl.BlockSpec((NCHUNK, ROW_BLK, HC), lambda i: (0, i, 0)),
            pl.BlockSpec((ROW_BLK, 1), lambda i: (i, 0)),
            pl.BlockSpec((1, H), lambda i: (0, 0)),
        ],
        out_specs=pl.BlockSpec((ROW_BLK, H), lambda i: (i, 0)),
        out_shape=jax.ShapeDtypeStruct((N, H), jnp.float32),
    )(s4, g4, dis, b2.reshape(1, H))


def kernel(x, edge_index, enc_W, enc_b, bn_gamma, bn_beta, W1, b1, W2, b2):
    ei = edge_index.astype(jnp.int32)
    col3d = ei[1].reshape(NC * NS, 1, DEG_PT)
    row3 = ei[0].reshape(NS, ITERS, EB)
    col3 = ei[1].reshape(NS, ITERS, EB)
    degp = _deg_call(col3d)
    dis = _dis_call(degp.reshape(NC * NS, N))
    g1 = _enc_call(x, enc_W, enc_b, bn_gamma, bn_beta, W1, dis)
    s1 = _scatter_call(g1, row3, col3)
    g2 = _mid_call(s1, g1, dis, b1, W2)
    s2 = _scatter_call(g2, row3, col3)
    return _fin_call(s2, g2, dis, b2)
